# Initial kernel scaffold; baseline (speedup 1.0000x reference)
import jax
import jax.numpy as jnp
from jax.experimental import pallas as pl


def kernel(input):
    raise NotImplementedError("write your pallas kernel here")


# speedup vs baseline: 1.1725x; 1.1725x over previous
import jax
import jax.numpy as jnp
from jax import lax
from jax.experimental import pallas as pl
from jax.experimental.pallas import tpu as pltpu
from jax.experimental.pallas import tpu_sc as plsc

_N = 65536
_D = 256
_NC = 2
_NS = 16
_CHUNK = _N // _NS
_NVREG = _CHUNK // 16
_RADIX = 256
_NB = _CHUNK // 128
_ROWS_W = _N // (_NC * _NS)
_WIN = 128
_NWIN = _ROWS_W // _WIN


def _sc_body(x_hbm, keys_hbm, out_hbm,
             a_k, a_v, b_k, b_v, hist_pub,
             ck, cv, rank, pos, hist_loc, hist_all, base, idxw, rows,
             gsem):
  c = lax.axis_index("c")
  s = lax.axis_index("s")
  t = s
  tbase = t * _CHUNK

  pltpu.sync_copy(keys_hbm.at[pl.ds(tbase, _CHUNK)], ck)

  def xf_body(j, _):
    off = pl.multiple_of(j * 16, 16)
    cv[pl.ds(off, 16)] = tbase + off + lax.iota(jnp.int32, 16)
    return 0

  lax.fori_loop(0, _NVREG, xf_body, 0)

  def run_pass(shift, load_from, dst_k, dst_v):
    if load_from is not None:
      src_k, src_v = load_from
      pltpu.sync_copy(src_k.at[pl.ds(tbase, _CHUNK)], ck)
      pltpu.sync_copy(src_v.at[pl.ds(tbase, _CHUNK)], cv)

    def z_body(i, _):
      hist_loc[pl.ds(pl.multiple_of(i * 16, 16), 16)] = jnp.zeros(
          (16,), jnp.int32)
      return 0

    lax.fori_loop(0, _RADIX // 16, z_body, 0)

    sh = jnp.int32(shift)

    def a_body(j, _):
      off = pl.multiple_of(j * 16, 16)
      ku = ck[pl.ds(off, 16)]
      d = lax.shift_right_logical(ku, sh) & jnp.int32(_RADIX - 1)
      old = plsc.load_gather(hist_loc, [d])
      cnt, last = plsc.scan_count(d)
      rank[pl.ds(off, 16)] = old + cnt - 1
      plsc.store_scatter(hist_loc, [d], old + cnt, mask=last)
      return 0

    lax.fori_loop(0, _NVREG, a_body, 0)

    pltpu.sync_copy(hist_loc, hist_pub.at[t])
    plsc.subcore_barrier()
    pltpu.sync_copy(hist_pub, hist_all)

    def b_body(g, carry):
      off = pl.multiple_of(g * 16, 16)
      colsum = jnp.zeros((16,), jnp.int32)
      myprefix = jnp.zeros((16,), jnp.int32)
      for w in range(_NS):
        row = hist_all[w, pl.ds(off, 16)]
        colsum = colsum + row
        m = jnp.where(jnp.int32(w) < t, jnp.int32(1), jnp.int32(0))
        myprefix = myprefix + row * m
      gs = plsc.cumsum(colsum)
      base[pl.ds(off, 16)] = gs - colsum + carry + myprefix
      return carry + jnp.sum(colsum)

    lax.fori_loop(0, _RADIX // 16, b_body, jnp.int32(0))

    def c_body(j, _):
      jb = pl.multiple_of(j * 128, 128)
      prow = pos.at[j]
      for u in range(8):
        off = pl.multiple_of(jb + u * 16, 16)
        ku = ck[pl.ds(off, 16)]
        d = lax.shift_right_logical(ku, sh) & jnp.int32(_RADIX - 1)
        p = plsc.load_gather(base, [d]) + rank[pl.ds(off, 16)]
        prow[pl.ds(u * 16, 16)] = p
      pltpu.sync_copy(ck.at[pl.ds(jb, 128)], dst_k.at[prow])
      pltpu.sync_copy(cv.at[pl.ds(jb, 128)], dst_v.at[prow])
      return 0

    lax.fori_loop(0, _NB, c_body, 0)
    plsc.subcore_barrier()

  run_pass(0, None, a_k, a_v)
  run_pass(8, (a_k, a_v), b_k, b_v)
  run_pass(16, (b_k, b_v), a_k, a_v)
  run_pass(24, (a_k, a_v), b_k, b_v)

  w = c * _NS + s
  obase = w * _ROWS_W
  for i in range(_NWIN):
    pltpu.sync_copy(b_v.at[pl.ds(obase + i * _WIN, _WIN)], idxw.at[i])

  desc = pltpu.async_copy(x_hbm.at[idxw.at[0]], rows.at[0], gsem)
  for i in range(_NWIN):
    desc.wait()
    if i + 1 < _NWIN:
      desc = pltpu.async_copy(x_hbm.at[idxw.at[i + 1]], rows.at[(i + 1) % 2],
                              gsem)
    pltpu.sync_copy(rows.at[i % 2], out_hbm.at[pl.ds(obase + i * _WIN, _WIN)])


@jax.jit
def _impl(x):
  f = x[:, 0]
  f = jnp.where(f == 0.0, jnp.float32(0.0), f)
  u = jax.lax.bitcast_convert_type(f, jnp.uint32)
  ku = jnp.where(u >> 31 == 1, ~u, u | jnp.uint32(0x80000000))
  keys = jax.lax.bitcast_convert_type(ku, jnp.int32)
  f = pl.kernel(
      _sc_body,
      out_type=jax.ShapeDtypeStruct((_N, _D), jnp.float32),
      mesh=plsc.VectorSubcoreMesh(core_axis_name="c", subcore_axis_name="s"),
      compiler_params=pltpu.CompilerParams(needs_layout_passes=False),
      scratch_types=[
          pltpu.VMEM_SHARED((_N,), jnp.int32),
          pltpu.VMEM_SHARED((_N,), jnp.int32),
          pltpu.VMEM_SHARED((_N,), jnp.int32),
          pltpu.VMEM_SHARED((_N,), jnp.int32),
          pltpu.VMEM_SHARED((_NS, _RADIX), jnp.int32),
          pltpu.VMEM((_CHUNK,), jnp.int32),
          pltpu.VMEM((_CHUNK,), jnp.int32),
          pltpu.VMEM((_CHUNK,), jnp.int32),
          pltpu.VMEM((_NB, 128), jnp.int32),
          pltpu.VMEM((_RADIX,), jnp.int32),
          pltpu.VMEM((_NS, _RADIX), jnp.int32),
          pltpu.VMEM((_RADIX,), jnp.int32),
          pltpu.VMEM((_NWIN, _WIN), jnp.int32),
          pltpu.VMEM((2, _WIN, _D), jnp.float32),
          pltpu.SemaphoreType.DMA,
      ],
  )
  return f(x, keys)


def kernel(input):
  return _impl(input)


# speedup vs baseline: 1.3723x; 1.1704x over previous
import jax
import jax.numpy as jnp
from jax import lax
from jax.experimental import pallas as pl
from jax.experimental.pallas import tpu as pltpu
from jax.experimental.pallas import tpu_sc as plsc

_N = 65536
_D = 256
_NC = 2
_NS = 16
_CHUNK = _N // _NS
_NVREG = _CHUNK // 16
_RADIX = 256
_NB = _CHUNK // 128
_ROWS_W = _N // (_NC * _NS)
_WIN = 128
_NWIN = _ROWS_W // _WIN


def _sc_body(x_hbm, keys_hbm, out_hbm,
             a_k, a_v, b_k, b_v, hist_pub,
             ck, cv, rank, pos, hist_loc, hist_all, base, idxw, rows,
             gsem, ssem):
  c = lax.axis_index("c")
  s = lax.axis_index("s")
  t = s
  tbase = t * _CHUNK

  pltpu.sync_copy(keys_hbm.at[pl.ds(tbase, _CHUNK)], ck)

  def xf_body(j, _):
    off = pl.multiple_of(j * 16, 16)
    cv[pl.ds(off, 16)] = tbase + off + lax.iota(jnp.int32, 16)
    return 0

  lax.fori_loop(0, _NVREG, xf_body, 0)

  def run_pass(shift, load_from, dst_k, dst_v):
    if load_from is not None:
      src_k, src_v = load_from
      d1 = pltpu.async_copy(src_k.at[pl.ds(tbase, _CHUNK)], ck, ssem)
      d2 = pltpu.async_copy(src_v.at[pl.ds(tbase, _CHUNK)], cv, ssem)
      d1.wait()
      d2.wait()

    def z_body(i, _):
      hist_loc[pl.ds(pl.multiple_of(i * 16, 16), 16)] = jnp.zeros(
          (16,), jnp.int32)
      return 0

    lax.fori_loop(0, _RADIX // 16, z_body, 0)

    sh = jnp.int32(shift)

    def a_body(j, _):
      off = pl.multiple_of(j * 16, 16)
      ku = ck[pl.ds(off, 16)]
      d = lax.shift_right_logical(ku, sh) & jnp.int32(_RADIX - 1)
      old = plsc.load_gather(hist_loc, [d])
      cnt, last = plsc.scan_count(d)
      rank[pl.ds(off, 16)] = old + cnt - 1
      plsc.store_scatter(hist_loc, [d], old + cnt, mask=last)
      return 0

    lax.fori_loop(0, _NVREG, a_body, 0, unroll=4)

    pltpu.sync_copy(hist_loc, hist_pub.at[t])
    plsc.subcore_barrier()
    pltpu.sync_copy(hist_pub, hist_all)

    def b_body(g, carry):
      off = pl.multiple_of(g * 16, 16)
      colsum = jnp.zeros((16,), jnp.int32)
      myprefix = jnp.zeros((16,), jnp.int32)
      for w in range(_NS):
        row = hist_all[w, pl.ds(off, 16)]
        colsum = colsum + row
        m = jnp.where(jnp.int32(w) < t, jnp.int32(1), jnp.int32(0))
        myprefix = myprefix + row * m
      gs = plsc.cumsum(colsum)
      base[pl.ds(off, 16)] = gs - colsum + carry + myprefix
      return carry + jnp.sum(colsum)

    lax.fori_loop(0, _RADIX // 16, b_body, jnp.int32(0))

    gb = 4

    def c_group(j):
      for u in range(gb):
        b = j * gb + u
        jb = pl.multiple_of(b * 128, 128)
        prow = pos.at[b]
        for u2 in range(8):
          off = pl.multiple_of(jb + u2 * 16, 16)
          ku = ck[pl.ds(off, 16)]
          d = lax.shift_right_logical(ku, sh) & jnp.int32(_RADIX - 1)
          p = plsc.load_gather(base, [d]) + rank[pl.ds(off, 16)]
          prow[pl.ds(u2 * 16, 16)] = p
        pltpu.async_copy(ck.at[pl.ds(jb, 128)], dst_k.at[prow], ssem)
        pltpu.async_copy(cv.at[pl.ds(jb, 128)], dst_v.at[prow], ssem)

    def c_drain(b_first):
      for u in range(gb):
        b = b_first + u
        jb = pl.multiple_of(b * 128, 128)
        pltpu.make_async_copy(
            ck.at[pl.ds(jb, 128)], dst_k.at[pos.at[b]], ssem).wait()
        pltpu.make_async_copy(
            cv.at[pl.ds(jb, 128)], dst_v.at[pos.at[b]], ssem).wait()

    def c_body(j, _):
      c_group(j)

      @pl.when(j > 0)
      def _():
        c_drain((j - 1) * gb)

      return 0

    lax.fori_loop(0, _NB // gb, c_body, 0)
    c_drain((_NB // gb - 1) * gb)
    plsc.subcore_barrier()

  run_pass(0, None, a_k, a_v)
  run_pass(8, (a_k, a_v), b_k, b_v)
  run_pass(16, (b_k, b_v), a_k, a_v)
  run_pass(24, (a_k, a_v), b_k, b_v)

  w = c * _NS + s
  obase = w * _ROWS_W
  for i in range(_NWIN):
    pltpu.sync_copy(b_v.at[pl.ds(obase + i * _WIN, _WIN)], idxw.at[i])

  desc = pltpu.async_copy(x_hbm.at[idxw.at[0]], rows.at[0], gsem)
  for i in range(_NWIN):
    desc.wait()
    if i + 1 < _NWIN:
      desc = pltpu.async_copy(x_hbm.at[idxw.at[i + 1]], rows.at[(i + 1) % 2],
                              gsem)
    pltpu.sync_copy(rows.at[i % 2], out_hbm.at[pl.ds(obase + i * _WIN, _WIN)])


@jax.jit
def _impl(x):
  f = x[:, 0]
  f = jnp.where(f == 0.0, jnp.float32(0.0), f)
  u = jax.lax.bitcast_convert_type(f, jnp.uint32)
  ku = jnp.where(u >> 31 == 1, ~u, u | jnp.uint32(0x80000000))
  keys = jax.lax.bitcast_convert_type(ku, jnp.int32)
  f = pl.kernel(
      _sc_body,
      out_type=jax.ShapeDtypeStruct((_N, _D), jnp.float32),
      mesh=plsc.VectorSubcoreMesh(core_axis_name="c", subcore_axis_name="s"),
      compiler_params=pltpu.CompilerParams(needs_layout_passes=False),
      scratch_types=[
          pltpu.VMEM_SHARED((_N,), jnp.int32),
          pltpu.VMEM_SHARED((_N,), jnp.int32),
          pltpu.VMEM_SHARED((_N,), jnp.int32),
          pltpu.VMEM_SHARED((_N,), jnp.int32),
          pltpu.VMEM_SHARED((_NS, _RADIX), jnp.int32),
          pltpu.VMEM((_CHUNK,), jnp.int32),
          pltpu.VMEM((_CHUNK,), jnp.int32),
          pltpu.VMEM((_CHUNK,), jnp.int32),
          pltpu.VMEM((_NB, 128), jnp.int32),
          pltpu.VMEM((_RADIX,), jnp.int32),
          pltpu.VMEM((_NS, _RADIX), jnp.int32),
          pltpu.VMEM((_RADIX,), jnp.int32),
          pltpu.VMEM((_NWIN, _WIN), jnp.int32),
          pltpu.VMEM((2, _WIN, _D), jnp.float32),
          pltpu.SemaphoreType.DMA,
          pltpu.SemaphoreType.DMA,
      ],
  )
  return f(x, keys)


def kernel(input):
  return _impl(input)
